# baseline (speedup 1.0000x reference)
import functools

import jax
import jax.numpy as jnp
from jax import lax
from jax.experimental import pallas as pl
from jax.experimental.pallas import tpu as pltpu
from jax.experimental.pallas import tpu_sc as plsc

B = 4096
WIN = 20
NEG = 20
NSYN = 8
NANT = 8
D = 128
DH = 64
NC, NS, L = 2, 16, 16
NW = NC * NS
BPW = B // NW
CCB = 16
PCB = 16
SCB = 32
EPS = 1e-10

_SDS = jax.ShapeDtypeStruct


def _mesh():
    return plsc.VectorSubcoreMesh(core_axis_name="c", subcore_axis_name="s",
                                  num_cores=NC, num_subcores=NS)


def _wid():
    return lax.axis_index("s") * NC + lax.axis_index("c")


def _db_loop(nch, gather, compute, bufs, sem0, sem1):
    gather(0, bufs[0], sem0).start()

    def pair(p2, _):
        ch0 = 2 * p2
        gather(ch0 + 1, bufs[1], sem1).start()
        gather(ch0, bufs[0], sem0).wait()
        compute(ch0, bufs[0])

        @pl.when(p2 + 1 < nch // 2)
        def _():
            gather(ch0 + 2, bufs[0], sem0).start()

        gather(ch0 + 1, bufs[1], sem1).wait()
        compute(ch0 + 1, bufs[1])
        return 0

    lax.fori_loop(0, nch // 2, pair, 0)


def _sc_ctxt_body(cix, embc, ctxt_out,
                  idx_v, rows0, rows1, ctxt_v, sem0, sem1):
    base = pl.multiple_of(_wid() * BPW, BPW)
    nch = BPW // CCB
    cch = CCB * WIN
    pltpu.sync_copy(cix.at[pl.ds(pl.multiple_of(base * WIN, 8), BPW * WIN)], idx_v)

    def gather(ch, buf, sem):
        return pltpu.make_async_copy(
            embc.at[idx_v.at[pl.ds(pl.multiple_of(ch * cch, 8), cch)]], buf, sem)

    def compute(ch, buf):
        def bb(b, c2):
            r0 = b * WIN
            for l in range(D // L):
                acc = buf[r0, pl.ds(l * L, L)]
                for j in range(1, WIN):
                    acc = acc + buf[r0 + j, pl.ds(l * L, L)]
                ctxt_v[ch * CCB + b, pl.ds(l * L, L)] = acc
            return c2
        lax.fori_loop(0, CCB, bb, 0)

    _db_loop(nch, gather, compute, (rows0, rows1), sem0, sem1)
    pltpu.sync_copy(ctxt_v, ctxt_out.at[pl.ds(pl.multiple_of(base, 8), BPW)])


@functools.cache
def _build_sc_ctxt():
    return pl.kernel(
        _sc_ctxt_body,
        out_type=_SDS((B, D), jnp.float32),
        mesh=_mesh(),
        scratch_types=[
            pltpu.VMEM((BPW * WIN,), jnp.int32),
            pltpu.VMEM((CCB * WIN, D), jnp.float32),
            pltpu.VMEM((CCB * WIN, D), jnp.float32),
            pltpu.VMEM((BPW, D), jnp.float32),
            pltpu.SemaphoreType.DMA,
            pltpu.SemaphoreType.DMA,
        ],
    )


def _sc_ctxt(*args):
    return _build_sc_ctxt()(*args)


def _sc_word_body(wix, six, aix, embi, dep, part_out, sdot, adot,
                  widx_v, wrows_v, sidx_v, srows0, srows1, sdot_v,
                  sem0, sem1, semw):
    del dep
    base = pl.multiple_of(_wid() * BPW, BPW)
    pltpu.sync_copy(wix.at[pl.ds(pl.multiple_of(base, 8), BPW)], widx_v)
    wdesc = pltpu.async_copy(embi.at[widx_v], wrows_v, semw)
    wdesc.wait()
    pltpu.sync_copy(wrows_v, part_out.at[pl.ds(pl.multiple_of(base, 8), BPW)])

    def sa_phase(ix, out):
        nch = BPW // SCB
        chr_ = SCB * NSYN
        orow = chr_ // 8
        pltpu.sync_copy(ix.at[pl.ds(pl.multiple_of(base * NSYN, 8), BPW * NSYN)],
                        sidx_v)

        def gather(ch, buf, sem):
            return pltpu.make_async_copy(
                embi.at[sidx_v.at[pl.ds(pl.multiple_of(ch * chr_, 8), chr_)]],
                buf, sem)

        def compute(ch, buf):
            def bb(b, c2):
                accs = [None] * NSYN
                for l in range(DH // L):
                    qv = wrows_v[ch * SCB + b, pl.ds(l * L, L)]
                    for j in range(NSYN):
                        prod = buf[b * NSYN + j, pl.ds(l * L, L)] * qv
                        accs[j] = prod if l == 0 else accs[j] + prod
                for j in range(NSYN):
                    f = b * NSYN + j
                    sdot_v[f // 8, pl.ds((f % 8) * L, L)] = accs[j]
                return c2
            lax.fori_loop(0, SCB, bb, 0)
            pltpu.sync_copy(
                sdot_v,
                out.at[pl.ds(pl.multiple_of((base + ch * SCB) * NSYN // 8, 8), orow)])

        _db_loop(nch, gather, compute, (srows0, srows1), sem0, sem1)

    sa_phase(six, sdot)
    sa_phase(aix, adot)


@functools.cache
def _build_sc_word():
    return pl.kernel(
        _sc_word_body,
        out_type=(_SDS((B, D), jnp.float32),
                  _SDS((B * NSYN * L // D, D), jnp.float32),
                  _SDS((B * NANT * L // D, D), jnp.float32)),
        mesh=_mesh(),
        scratch_types=[
            pltpu.VMEM((BPW,), jnp.int32),
            pltpu.VMEM((BPW, D), jnp.float32),
            pltpu.VMEM((BPW * NSYN,), jnp.int32),
            pltpu.VMEM((SCB * NSYN, D), jnp.float32),
            pltpu.VMEM((SCB * NSYN, D), jnp.float32),
            pltpu.VMEM((SCB * NSYN * L // D, D), jnp.float32),
            pltpu.SemaphoreType.DMA,
            pltpu.SemaphoreType.DMA,
            pltpu.SemaphoreType.DMA,
        ],
    )


def _sc_word(*args):
    return _build_sc_word()(*args)


def _sc_dots_body(pix, nix, embo, inp,
                  pdot, ndot,
                  inp_v, idx_v, rows0, rows1, dot_v, sem0, sem1):
    base = pl.multiple_of(_wid() * BPW, BPW)
    pltpu.sync_copy(inp.at[pl.ds(pl.multiple_of(base, 8), BPW)], inp_v)
    nch = BPW // PCB
    chr_ = PCB * WIN
    orow = PCB * WIN * L // D

    def phase(ix, out):
        pltpu.sync_copy(ix.at[pl.ds(pl.multiple_of(base * WIN, 8), BPW * WIN)], idx_v)

        def gather(ch, buf, sem):
            return pltpu.make_async_copy(
                embo.at[idx_v.at[pl.ds(pl.multiple_of(ch * chr_, 8), chr_)]], buf, sem)

        def compute(ch, buf):
            def bb(b, c2):
                accs = [None] * WIN
                for l in range(D // L):
                    qv = inp_v[ch * PCB + b, pl.ds(l * L, L)]
                    for j in range(WIN):
                        prod = buf[b * WIN + j, pl.ds(l * L, L)] * qv
                        accs[j] = prod if l == 0 else accs[j] + prod
                for j in range(WIN):
                    f = b * WIN + j
                    dot_v[f // 8, pl.ds((f % 8) * L, L)] = accs[j]
                return c2
            lax.fori_loop(0, PCB, bb, 0)
            pltpu.sync_copy(
                dot_v,
                out.at[pl.ds(pl.multiple_of((base + ch * PCB) * WIN // 8, 8), orow)])

        _db_loop(nch, gather, compute, (rows0, rows1), sem0, sem1)

    phase(pix, pdot)
    phase(nix, ndot)


@functools.cache
def _build_sc_dots():
    return pl.kernel(
        _sc_dots_body,
        out_type=(_SDS((B * WIN * L // D, D), jnp.float32),
                  _SDS((B * NEG * L // D, D), jnp.float32)),
        mesh=_mesh(),
        scratch_types=[
            pltpu.VMEM((BPW, D), jnp.float32),
            pltpu.VMEM((BPW * WIN,), jnp.int32),
            pltpu.VMEM((PCB * WIN, D), jnp.float32),
            pltpu.VMEM((PCB * WIN, D), jnp.float32),
            pltpu.VMEM((PCB * WIN * L // D, D), jnp.float32),
            pltpu.SemaphoreType.DMA,
            pltpu.SemaphoreType.DMA,
        ],
    )


def _sc_dots(*args):
    return _build_sc_dots()(*args)


def _softplus(x):
    return jnp.maximum(x, 0.0) + jnp.log1p(jnp.exp(-jnp.abs(x)))


def _lane_group_matrix():
    return (lax.broadcasted_iota(jnp.int32, (D, D // L), 0) // L ==
            lax.broadcasted_iota(jnp.int32, (D, D // L), 1)).astype(jnp.float32)


def _tc_mlp_body(ctxt_ref, part_ref, w0, b0, w1, b1, wmu, bmu, wlv, blv, rnd,
                 inp_ref, kl_ref):
    ctxt = ctxt_ref[...]
    enc = jnp.tanh(ctxt @ w0[...] + b0[...])
    enc = jnp.tanh(enc @ w1[...] + b1[...])
    mu = enc @ wmu[...] + bmu[...]
    logvar = enc @ wlv[...] + blv[...]
    sigma = jnp.exp(logvar * 0.5)
    z = mu + sigma * rnd[...]
    inp_ref[...] = jnp.concatenate([z, part_ref[:, :DH]], axis=1)
    kl = -0.5 * jnp.sum(1.0 + logvar - mu * mu - jnp.exp(logvar))
    kl_ref[...] = kl.reshape(1, 1)


def _tc_mlp(*args):
    return pl.pallas_call(
        _tc_mlp_body,
        out_shape=(_SDS((B, D), jnp.float32), _SDS((1, 1), jnp.float32)),
    )(*args)


def _tc_sa_body(sd, ad, ms, ma, ss_ref, asc_ref):
    g = _lane_group_matrix()
    s = sd[...] @ g + EPS
    ss_ref[...] = jnp.sum(ms[...] * _softplus(-s)).reshape(1, 1)
    a = ad[...] @ g - EPS
    asc_ref[...] = jnp.sum(ma[...] * _softplus(a)).reshape(1, 1)


def _tc_sa(*args):
    return pl.pallas_call(
        _tc_sa_body,
        out_shape=(_SDS((1, 1), jnp.float32),) * 2,
    )(*args)


def _tc_loss_body(pd, nd, ps, ns):
    g = _lane_group_matrix()
    p = pd[...] @ g + EPS
    ps[...] = jnp.sum(_softplus(-p)).reshape(1, 1)
    n = nd[...] @ g - EPS
    ns[...] = jnp.sum(_softplus(n)).reshape(1, 1)


def _tc_loss(*args):
    return pl.pallas_call(
        _tc_loss_body,
        out_shape=(_SDS((1, 1), jnp.float32),) * 2,
    )(*args)


def kernel(w_ix, p_ix, c_ix, neg_ix, syn_ix, ms_ix, ant_ix, ma_ix,
           emb_i, emb_o, emb_c, W0, b0, W1, b1, Wmu, bmu, Wlv, blv):
    ii = lambda a: a.reshape(-1).astype(jnp.int32)
    rnd = jax.random.normal(jax.random.key(42), (B, DH), dtype=jnp.float32)
    embi_pad = jnp.pad(emb_i, ((0, 0), (0, D - DH)))
    ctxt = _sc_ctxt(ii(c_ix), emb_c)
    part, sdot, adot = _sc_word(ii(w_ix), ii(syn_ix), ii(ant_ix), embi_pad, ctxt)
    inp, kl_raw = _tc_mlp(
        ctxt, part, W0, b0.reshape(1, D), W1, b1.reshape(1, D),
        Wmu, bmu.reshape(1, DH), Wlv, blv.reshape(1, DH), rnd)
    pdot, ndot = _sc_dots(ii(p_ix), ii(neg_ix), emb_o, inp)
    ss, asc = _tc_sa(sdot, adot, ms_ix, ma_ix)
    ps, ns = _tc_loss(pdot, ndot)
    p_score = ps[0, 0]
    n_score = ns[0, 0]
    syn_score = ss[0, 0]
    ant_score = asc[0, 0]
    kl_loss = kl_raw[0, 0] / float(WIN * NEG)
    decoder_loss = p_score + n_score + syn_score + ant_score
    loss = kl_loss + decoder_loss
    inv = 1.0 / B
    return (loss * inv, kl_loss * inv, decoder_loss * inv, p_score * inv,
            n_score * inv, syn_score * inv, ant_score * inv)


# speedup vs baseline: 1.0649x; 1.0649x over previous
import functools

import jax
import jax.numpy as jnp
from jax import lax
from jax.experimental import pallas as pl
from jax.experimental.pallas import tpu as pltpu
from jax.experimental.pallas import tpu_sc as plsc

B = 4096
WIN = 20
NEG = 20
NSYN = 8
NANT = 8
D = 128
DH = 64
NC, NS, L = 2, 16, 16
NW = NC * NS
BPW = B // NW
CCB = 16
PCB = 16
SCB = 32
EPS = 1e-10

_SDS = jax.ShapeDtypeStruct


def _mesh():
    return plsc.VectorSubcoreMesh(core_axis_name="c", subcore_axis_name="s",
                                  num_cores=NC, num_subcores=NS)


def _wid():
    return lax.axis_index("s") * NC + lax.axis_index("c")


def _db_loop(nch, gather, compute, bufs, sem0, sem1):
    gather(0, bufs[0], sem0).start()

    def pair(p2, _):
        ch0 = 2 * p2
        gather(ch0 + 1, bufs[1], sem1).start()
        gather(ch0, bufs[0], sem0).wait()
        compute(ch0, bufs[0])

        @pl.when(p2 + 1 < nch // 2)
        def _():
            gather(ch0 + 2, bufs[0], sem0).start()

        gather(ch0 + 1, bufs[1], sem1).wait()
        compute(ch0 + 1, bufs[1])
        return 0

    lax.fori_loop(0, nch // 2, pair, 0)


def _sc_ctxt_body(cix, embc, ctxt_out,
                  idx_v, rows0, rows1, ctxt_v, sem0, sem1):
    base = pl.multiple_of(_wid() * BPW, BPW)
    nch = BPW // CCB
    cch = CCB * WIN
    pltpu.sync_copy(cix.at[pl.ds(pl.multiple_of(base * WIN, 8), BPW * WIN)], idx_v)

    def gather(ch, buf, sem):
        return pltpu.make_async_copy(
            embc.at[idx_v.at[pl.ds(pl.multiple_of(ch * cch, 8), cch)]], buf, sem)

    def compute(ch, buf):
        def bb(b, c2):
            r0 = b * WIN
            for l in range(D // L):
                acc = buf[r0, pl.ds(l * L, L)]
                for j in range(1, WIN):
                    acc = acc + buf[r0 + j, pl.ds(l * L, L)]
                ctxt_v[ch * CCB + b, pl.ds(l * L, L)] = acc
            return c2
        lax.fori_loop(0, CCB, bb, 0)

    _db_loop(nch, gather, compute, (rows0, rows1), sem0, sem1)
    pltpu.sync_copy(ctxt_v, ctxt_out.at[pl.ds(pl.multiple_of(base, 8), BPW)])


@functools.cache
def _build_sc_ctxt():
    return pl.kernel(
        _sc_ctxt_body,
        out_type=_SDS((B, D), jnp.float32),
        mesh=_mesh(),
        scratch_types=[
            pltpu.VMEM((BPW * WIN,), jnp.int32),
            pltpu.VMEM((CCB * WIN, D), jnp.float32),
            pltpu.VMEM((CCB * WIN, D), jnp.float32),
            pltpu.VMEM((BPW, D), jnp.float32),
            pltpu.SemaphoreType.DMA,
            pltpu.SemaphoreType.DMA,
        ],
    )


def _sc_ctxt(*args):
    return _build_sc_ctxt()(*args)


def _sc_word_body(wix, six, aix, embi, dep, part_out, sdot, adot,
                  widx_v, wrows_v, sidx_v, srows0, srows1, sdot_v,
                  sem0, sem1, semw):
    del dep
    base = pl.multiple_of(_wid() * BPW, BPW)
    pltpu.sync_copy(wix.at[pl.ds(pl.multiple_of(base, 8), BPW)], widx_v)
    wdesc = pltpu.async_copy(embi.at[widx_v], wrows_v, semw)
    wdesc.wait()
    pltpu.sync_copy(wrows_v, part_out.at[pl.ds(pl.multiple_of(base, 8), BPW)])

    def sa_phase(ix, out):
        nch = BPW // SCB
        chr_ = SCB * NSYN
        orow = chr_ // 8
        pltpu.sync_copy(ix.at[pl.ds(pl.multiple_of(base * NSYN, 8), BPW * NSYN)],
                        sidx_v)

        def gather(ch, buf, sem):
            return pltpu.make_async_copy(
                embi.at[sidx_v.at[pl.ds(pl.multiple_of(ch * chr_, 8), chr_)]],
                buf, sem)

        def compute(ch, buf):
            def bb(b, c2):
                accs = [None] * NSYN
                for l in range(DH // L):
                    qv = wrows_v[ch * SCB + b, pl.ds(l * L, L)]
                    for j in range(NSYN):
                        prod = buf[b * NSYN + j, pl.ds(l * L, L)] * qv
                        accs[j] = prod if l == 0 else accs[j] + prod
                for j in range(NSYN):
                    f = b * NSYN + j
                    sdot_v[f // 8, pl.ds((f % 8) * L, L)] = accs[j]
                return c2
            lax.fori_loop(0, SCB, bb, 0)
            pltpu.sync_copy(
                sdot_v,
                out.at[pl.ds(pl.multiple_of((base + ch * SCB) * NSYN // 8, 8), orow)])

        _db_loop(nch, gather, compute, (srows0, srows1), sem0, sem1)

    sa_phase(six, sdot)
    sa_phase(aix, adot)


@functools.cache
def _build_sc_word():
    return pl.kernel(
        _sc_word_body,
        out_type=(_SDS((B, DH), jnp.float32),
                  _SDS((B * NSYN * L // D, D), jnp.float32),
                  _SDS((B * NANT * L // D, D), jnp.float32)),
        mesh=_mesh(),
        scratch_types=[
            pltpu.VMEM((BPW,), jnp.int32),
            pltpu.VMEM((BPW, DH), jnp.float32),
            pltpu.VMEM((BPW * NSYN,), jnp.int32),
            pltpu.VMEM((SCB * NSYN, DH), jnp.float32),
            pltpu.VMEM((SCB * NSYN, DH), jnp.float32),
            pltpu.VMEM((SCB * NSYN * L // D, D), jnp.float32),
            pltpu.SemaphoreType.DMA,
            pltpu.SemaphoreType.DMA,
            pltpu.SemaphoreType.DMA,
        ],
        compiler_params=pltpu.CompilerParams(use_tc_tiling_on_sc=False),
    )


def _sc_word(*args):
    return _build_sc_word()(*args)


def _sc_dots_body(pix, nix, embo, inp,
                  pdot, ndot,
                  inp_v, idx_v, rows0, rows1, dot_v, sem0, sem1):
    base = pl.multiple_of(_wid() * BPW, BPW)
    pltpu.sync_copy(inp.at[pl.ds(pl.multiple_of(base, 8), BPW)], inp_v)
    nch = BPW // PCB
    chr_ = PCB * WIN
    orow = PCB * WIN * L // D

    def phase(ix, out):
        pltpu.sync_copy(ix.at[pl.ds(pl.multiple_of(base * WIN, 8), BPW * WIN)], idx_v)

        def gather(ch, buf, sem):
            return pltpu.make_async_copy(
                embo.at[idx_v.at[pl.ds(pl.multiple_of(ch * chr_, 8), chr_)]], buf, sem)

        def compute(ch, buf):
            def bb(b, c2):
                accs = [None] * WIN
                for l in range(D // L):
                    qv = inp_v[ch * PCB + b, pl.ds(l * L, L)]
                    for j in range(WIN):
                        prod = buf[b * WIN + j, pl.ds(l * L, L)] * qv
                        accs[j] = prod if l == 0 else accs[j] + prod
                for j in range(WIN):
                    f = b * WIN + j
                    dot_v[f // 8, pl.ds((f % 8) * L, L)] = accs[j]
                return c2
            lax.fori_loop(0, PCB, bb, 0)
            pltpu.sync_copy(
                dot_v,
                out.at[pl.ds(pl.multiple_of((base + ch * PCB) * WIN // 8, 8), orow)])

        _db_loop(nch, gather, compute, (rows0, rows1), sem0, sem1)

    phase(pix, pdot)
    phase(nix, ndot)


@functools.cache
def _build_sc_dots():
    return pl.kernel(
        _sc_dots_body,
        out_type=(_SDS((B * WIN * L // D, D), jnp.float32),
                  _SDS((B * NEG * L // D, D), jnp.float32)),
        mesh=_mesh(),
        scratch_types=[
            pltpu.VMEM((BPW, D), jnp.float32),
            pltpu.VMEM((BPW * WIN,), jnp.int32),
            pltpu.VMEM((PCB * WIN, D), jnp.float32),
            pltpu.VMEM((PCB * WIN, D), jnp.float32),
            pltpu.VMEM((PCB * WIN * L // D, D), jnp.float32),
            pltpu.SemaphoreType.DMA,
            pltpu.SemaphoreType.DMA,
        ],
    )


def _sc_dots(*args):
    return _build_sc_dots()(*args)


def _softplus(x):
    return jnp.maximum(x, 0.0) + jnp.log1p(jnp.exp(-jnp.abs(x)))


def _lane_group_matrix():
    return (lax.broadcasted_iota(jnp.int32, (D, D // L), 0) // L ==
            lax.broadcasted_iota(jnp.int32, (D, D // L), 1)).astype(jnp.float32)


def _tc_mlp_body(ctxt_ref, part_ref, w0, b0, w1, b1, wmu, bmu, wlv, blv, rnd,
                 inp_ref, kl_ref):
    ctxt = ctxt_ref[...]
    enc = jnp.tanh(ctxt @ w0[...] + b0[...])
    enc = jnp.tanh(enc @ w1[...] + b1[...])
    mu = enc @ wmu[...] + bmu[...]
    logvar = enc @ wlv[...] + blv[...]
    sigma = jnp.exp(logvar * 0.5)
    z = mu + sigma * rnd[...]
    inp_ref[...] = jnp.concatenate([z, part_ref[...]], axis=1)
    kl = -0.5 * jnp.sum(1.0 + logvar - mu * mu - jnp.exp(logvar))
    kl_ref[...] = kl.reshape(1, 1)


def _tc_mlp(*args):
    return pl.pallas_call(
        _tc_mlp_body,
        out_shape=(_SDS((B, D), jnp.float32), _SDS((1, 1), jnp.float32)),
    )(*args)


def _tc_sa_body(sd, ad, ms, ma, ss_ref, asc_ref):
    g = _lane_group_matrix()
    s = sd[...] @ g + EPS
    ss_ref[...] = jnp.sum(ms[...] * _softplus(-s)).reshape(1, 1)
    a = ad[...] @ g - EPS
    asc_ref[...] = jnp.sum(ma[...] * _softplus(a)).reshape(1, 1)


def _tc_sa(*args):
    return pl.pallas_call(
        _tc_sa_body,
        out_shape=(_SDS((1, 1), jnp.float32),) * 2,
    )(*args)


def _tc_loss_body(pd, nd, ps, ns):
    g = _lane_group_matrix()
    p = pd[...] @ g + EPS
    ps[...] = jnp.sum(_softplus(-p)).reshape(1, 1)
    n = nd[...] @ g - EPS
    ns[...] = jnp.sum(_softplus(n)).reshape(1, 1)


def _tc_loss(*args):
    return pl.pallas_call(
        _tc_loss_body,
        out_shape=(_SDS((1, 1), jnp.float32),) * 2,
    )(*args)


def kernel(w_ix, p_ix, c_ix, neg_ix, syn_ix, ms_ix, ant_ix, ma_ix,
           emb_i, emb_o, emb_c, W0, b0, W1, b1, Wmu, bmu, Wlv, blv):
    ii = lambda a: a.reshape(-1).astype(jnp.int32)
    rnd = jax.random.normal(jax.random.key(42), (B, DH), dtype=jnp.float32)
    ctxt = _sc_ctxt(ii(c_ix), emb_c)
    part, sdot, adot = _sc_word(ii(w_ix), ii(syn_ix), ii(ant_ix), emb_i, ctxt)
    inp, kl_raw = _tc_mlp(
        ctxt, part, W0, b0.reshape(1, D), W1, b1.reshape(1, D),
        Wmu, bmu.reshape(1, DH), Wlv, blv.reshape(1, DH), rnd)
    pdot, ndot = _sc_dots(ii(p_ix), ii(neg_ix), emb_o, inp)
    ss, asc = _tc_sa(sdot, adot, ms_ix, ma_ix)
    ps, ns = _tc_loss(pdot, ndot)
    p_score = ps[0, 0]
    n_score = ns[0, 0]
    syn_score = ss[0, 0]
    ant_score = asc[0, 0]
    kl_loss = kl_raw[0, 0] / float(WIN * NEG)
    decoder_loss = p_score + n_score + syn_score + ant_score
    loss = kl_loss + decoder_loss
    inv = 1.0 / B
    return (loss * inv, kl_loss * inv, decoder_loss * inv, p_score * inv,
            n_score * inv, syn_score * inv, ant_score * inv)


# speedup vs baseline: 1.0685x; 1.0034x over previous
import functools

import jax
import jax.numpy as jnp
from jax import lax
from jax.experimental import pallas as pl
from jax.experimental.pallas import tpu as pltpu
from jax.experimental.pallas import tpu_sc as plsc

B = 4096
WIN = 20
NEG = 20
NSYN = 8
NANT = 8
D = 128
DH = 64
NC, NS, L = 2, 16, 16
NW = NC * NS
BPW = B // NW
CCB = 16
PCB = 16
SCB = 32
EPS = 1e-10

_SDS = jax.ShapeDtypeStruct


def _mesh():
    return plsc.VectorSubcoreMesh(core_axis_name="c", subcore_axis_name="s",
                                  num_cores=NC, num_subcores=NS)


def _wid():
    return lax.axis_index("s") * NC + lax.axis_index("c")


def _db_loop(nch, gather, compute, bufs, sem0, sem1):
    gather(0, bufs[0], sem0).start()

    def pair(p2, _):
        ch0 = 2 * p2
        gather(ch0 + 1, bufs[1], sem1).start()
        gather(ch0, bufs[0], sem0).wait()
        compute(ch0, bufs[0])

        @pl.when(p2 + 1 < nch // 2)
        def _():
            gather(ch0 + 2, bufs[0], sem0).start()

        gather(ch0 + 1, bufs[1], sem1).wait()
        compute(ch0 + 1, bufs[1])
        return 0

    lax.fori_loop(0, nch // 2, pair, 0)


def _sc_ctxt_body(cix, embc, ctxt_out,
                  idx_v, rows0, rows1, ctxt_v, sem0, sem1):
    base = pl.multiple_of(_wid() * BPW, BPW)
    nch = BPW // CCB
    cch = CCB * WIN
    pltpu.sync_copy(cix.at[pl.ds(pl.multiple_of(base * WIN, 8), BPW * WIN)], idx_v)

    def gather(ch, buf, sem):
        return pltpu.make_async_copy(
            embc.at[idx_v.at[pl.ds(pl.multiple_of(ch * cch, 8), cch)]], buf, sem)

    def compute(ch, buf):
        def bb(b, c2):
            r0 = b * WIN
            for l in range(D // L):
                acc = buf[r0, pl.ds(l * L, L)]
                for j in range(1, WIN):
                    acc = acc + buf[r0 + j, pl.ds(l * L, L)]
                ctxt_v[ch * CCB + b, pl.ds(l * L, L)] = acc
            return c2
        lax.fori_loop(0, CCB, bb, 0)

    _db_loop(nch, gather, compute, (rows0, rows1), sem0, sem1)
    pltpu.sync_copy(ctxt_v, ctxt_out.at[pl.ds(pl.multiple_of(base, 8), BPW)])


@functools.cache
def _build_sc_ctxt():
    return pl.kernel(
        _sc_ctxt_body,
        out_type=_SDS((B, D), jnp.float32),
        mesh=_mesh(),
        scratch_types=[
            pltpu.VMEM((BPW * WIN,), jnp.int32),
            pltpu.VMEM((CCB * WIN, D), jnp.float32),
            pltpu.VMEM((CCB * WIN, D), jnp.float32),
            pltpu.VMEM((BPW, D), jnp.float32),
            pltpu.SemaphoreType.DMA,
            pltpu.SemaphoreType.DMA,
        ],
    )


def _sc_ctxt(*args):
    return _build_sc_ctxt()(*args)


def _sc_word_body(wix, six, aix, embi, dep, part_out, sdot, adot,
                  widx_v, wrows_v, pinp_v, sidx_v, srows0, srows1, sdot_v,
                  sem0, sem1, semw):
    del dep
    base = pl.multiple_of(_wid() * BPW, BPW)
    pltpu.sync_copy(wix.at[pl.ds(pl.multiple_of(base, 8), BPW)], widx_v)
    wdesc = pltpu.async_copy(embi.at[widx_v], wrows_v, semw)
    wdesc.wait()
    zv = jnp.zeros((L,), jnp.float32)

    def prow(r, c2):
        for l in range(DH // L):
            pinp_v[r, pl.ds(l * L, L)] = zv
            pinp_v[r, pl.ds(DH + l * L, L)] = wrows_v[r, pl.ds(l * L, L)]
        return c2

    lax.fori_loop(0, BPW, prow, 0)
    pltpu.sync_copy(pinp_v, part_out.at[pl.ds(pl.multiple_of(base, 8), BPW)])

    def sa_phase(ix, out):
        nch = BPW // SCB
        chr_ = SCB * NSYN
        orow = chr_ // 8
        pltpu.sync_copy(ix.at[pl.ds(pl.multiple_of(base * NSYN, 8), BPW * NSYN)],
                        sidx_v)

        def gather(ch, buf, sem):
            return pltpu.make_async_copy(
                embi.at[sidx_v.at[pl.ds(pl.multiple_of(ch * chr_, 8), chr_)]],
                buf, sem)

        def compute(ch, buf):
            def bb(b, c2):
                accs = [None] * NSYN
                for l in range(DH // L):
                    qv = wrows_v[ch * SCB + b, pl.ds(l * L, L)]
                    for j in range(NSYN):
                        prod = buf[b * NSYN + j, pl.ds(l * L, L)] * qv
                        accs[j] = prod if l == 0 else accs[j] + prod
                for j in range(NSYN):
                    f = b * NSYN + j
                    sdot_v[f // 8, pl.ds((f % 8) * L, L)] = accs[j]
                return c2
            lax.fori_loop(0, SCB, bb, 0)
            pltpu.sync_copy(
                sdot_v,
                out.at[pl.ds(pl.multiple_of((base + ch * SCB) * NSYN // 8, 8), orow)])

        _db_loop(nch, gather, compute, (srows0, srows1), sem0, sem1)

    sa_phase(six, sdot)
    sa_phase(aix, adot)


@functools.cache
def _build_sc_word():
    return pl.kernel(
        _sc_word_body,
        out_type=(_SDS((B, D), jnp.float32),
                  _SDS((B * NSYN * L // D, D), jnp.float32),
                  _SDS((B * NANT * L // D, D), jnp.float32)),
        mesh=_mesh(),
        scratch_types=[
            pltpu.VMEM((BPW,), jnp.int32),
            pltpu.VMEM((BPW, DH), jnp.float32),
            pltpu.VMEM((BPW, D), jnp.float32),
            pltpu.VMEM((BPW * NSYN,), jnp.int32),
            pltpu.VMEM((SCB * NSYN, DH), jnp.float32),
            pltpu.VMEM((SCB * NSYN, DH), jnp.float32),
            pltpu.VMEM((SCB * NSYN * L // D, D), jnp.float32),
            pltpu.SemaphoreType.DMA,
            pltpu.SemaphoreType.DMA,
            pltpu.SemaphoreType.DMA,
        ],
        compiler_params=pltpu.CompilerParams(use_tc_tiling_on_sc=False),
    )


def _sc_word(*args):
    return _build_sc_word()(*args)


def _sc_dots_body(pix, nix, embo, zinp, pinp,
                  pdot, ndot,
                  inp_v, pq_v, idx_v, rows0, rows1, dot_v, sem0, sem1):
    base = pl.multiple_of(_wid() * BPW, BPW)
    pltpu.sync_copy(zinp.at[pl.ds(pl.multiple_of(base, 8), BPW)], inp_v)
    pltpu.sync_copy(pinp.at[pl.ds(pl.multiple_of(base, 8), BPW)], pq_v)

    def qrow(r, c2):
        for l in range(D // L):
            inp_v[r, pl.ds(l * L, L)] = (inp_v[r, pl.ds(l * L, L)] +
                                         pq_v[r, pl.ds(l * L, L)])
        return c2

    lax.fori_loop(0, BPW, qrow, 0)
    nch = BPW // PCB
    chr_ = PCB * WIN
    orow = PCB * WIN * L // D

    def phase(ix, out):
        pltpu.sync_copy(ix.at[pl.ds(pl.multiple_of(base * WIN, 8), BPW * WIN)], idx_v)

        def gather(ch, buf, sem):
            return pltpu.make_async_copy(
                embo.at[idx_v.at[pl.ds(pl.multiple_of(ch * chr_, 8), chr_)]], buf, sem)

        def compute(ch, buf):
            def bb(b, c2):
                accs = [None] * WIN
                for l in range(D // L):
                    qv = inp_v[ch * PCB + b, pl.ds(l * L, L)]
                    for j in range(WIN):
                        prod = buf[b * WIN + j, pl.ds(l * L, L)] * qv
                        accs[j] = prod if l == 0 else accs[j] + prod
                for j in range(WIN):
                    f = b * WIN + j
                    dot_v[f // 8, pl.ds((f % 8) * L, L)] = accs[j]
                return c2
            lax.fori_loop(0, PCB, bb, 0)
            pltpu.sync_copy(
                dot_v,
                out.at[pl.ds(pl.multiple_of((base + ch * PCB) * WIN // 8, 8), orow)])

        _db_loop(nch, gather, compute, (rows0, rows1), sem0, sem1)

    phase(pix, pdot)
    phase(nix, ndot)


@functools.cache
def _build_sc_dots():
    return pl.kernel(
        _sc_dots_body,
        out_type=(_SDS((B * WIN * L // D, D), jnp.float32),
                  _SDS((B * NEG * L // D, D), jnp.float32)),
        mesh=_mesh(),
        scratch_types=[
            pltpu.VMEM((BPW, D), jnp.float32),
            pltpu.VMEM((BPW, D), jnp.float32),
            pltpu.VMEM((BPW * WIN,), jnp.int32),
            pltpu.VMEM((PCB * WIN, D), jnp.float32),
            pltpu.VMEM((PCB * WIN, D), jnp.float32),
            pltpu.VMEM((PCB * WIN * L // D, D), jnp.float32),
            pltpu.SemaphoreType.DMA,
            pltpu.SemaphoreType.DMA,
        ],
    )


def _sc_dots(*args):
    return _build_sc_dots()(*args)


def _softplus(x):
    return jnp.maximum(x, 0.0) + jnp.log1p(jnp.exp(-jnp.abs(x)))


def _lane_group_matrix():
    return (lax.broadcasted_iota(jnp.int32, (D, D // L), 0) // L ==
            lax.broadcasted_iota(jnp.int32, (D, D // L), 1)).astype(jnp.float32)


def _tc_mlp_body(ctxt_ref, w0, b0, w1, b1, wmu, bmu, wlv, blv, rnd,
                 inp_ref, kl_ref):
    ctxt = ctxt_ref[...]
    enc = jnp.tanh(ctxt @ w0[...] + b0[...])
    enc = jnp.tanh(enc @ w1[...] + b1[...])
    mu = enc @ wmu[...] + bmu[...]
    logvar = enc @ wlv[...] + blv[...]
    sigma = jnp.exp(logvar * 0.5)
    z = mu + sigma * rnd[...]
    inp_ref[...] = jnp.concatenate([z, jnp.zeros_like(z)], axis=1)
    kl = -0.5 * jnp.sum(1.0 + logvar - mu * mu - jnp.exp(logvar))
    kl_ref[...] = kl.reshape(1, 1)


def _tc_mlp(*args):
    return pl.pallas_call(
        _tc_mlp_body,
        out_shape=(_SDS((B, D), jnp.float32), _SDS((1, 1), jnp.float32)),
    )(*args)


def _tc_sa_body(sd, ad, ms, ma, ss_ref, asc_ref):
    g = _lane_group_matrix()
    s = sd[...] @ g + EPS
    ss_ref[...] = jnp.sum(ms[...] * _softplus(-s)).reshape(1, 1)
    a = ad[...] @ g - EPS
    asc_ref[...] = jnp.sum(ma[...] * _softplus(a)).reshape(1, 1)


def _tc_sa(*args):
    return pl.pallas_call(
        _tc_sa_body,
        out_shape=(_SDS((1, 1), jnp.float32),) * 2,
    )(*args)


def _tc_loss_body(pd, nd, ps, ns):
    g = _lane_group_matrix()
    p = pd[...] @ g + EPS
    ps[...] = jnp.sum(_softplus(-p)).reshape(1, 1)
    n = nd[...] @ g - EPS
    ns[...] = jnp.sum(_softplus(n)).reshape(1, 1)


def _tc_loss(*args):
    return pl.pallas_call(
        _tc_loss_body,
        out_shape=(_SDS((1, 1), jnp.float32),) * 2,
    )(*args)


def kernel(w_ix, p_ix, c_ix, neg_ix, syn_ix, ms_ix, ant_ix, ma_ix,
           emb_i, emb_o, emb_c, W0, b0, W1, b1, Wmu, bmu, Wlv, blv):
    ii = lambda a: a.reshape(-1).astype(jnp.int32)
    rnd = jax.random.normal(jax.random.key(42), (B, DH), dtype=jnp.float32)
    ctxt = _sc_ctxt(ii(c_ix), emb_c)
    pinp, sdot, adot = _sc_word(ii(w_ix), ii(syn_ix), ii(ant_ix), emb_i, ctxt)
    zinp, kl_raw = _tc_mlp(
        ctxt, W0, b0.reshape(1, D), W1, b1.reshape(1, D),
        Wmu, bmu.reshape(1, DH), Wlv, blv.reshape(1, DH), rnd)
    pdot, ndot = _sc_dots(ii(p_ix), ii(neg_ix), emb_o, zinp, pinp)
    ss, asc = _tc_sa(sdot, adot, ms_ix, ma_ix)
    ps, ns = _tc_loss(pdot, ndot)
    p_score = ps[0, 0]
    n_score = ns[0, 0]
    syn_score = ss[0, 0]
    ant_score = asc[0, 0]
    kl_loss = kl_raw[0, 0] / float(WIN * NEG)
    decoder_loss = p_score + n_score + syn_score + ant_score
    loss = kl_loss + decoder_loss
    inv = 1.0 / B
    return (loss * inv, kl_loss * inv, decoder_loss * inv, p_score * inv,
            n_score * inv, syn_score * inv, ant_score * inv)
